# baseline (speedup 1.0000x reference)
import functools

import jax
import jax.numpy as jnp
from jax import lax
from jax.experimental import pallas as pl
from jax.experimental.pallas import tpu as pltpu
from jax.experimental.pallas import tpu_sc as plsc

_N = 22
_B = 1024
_D = 32
_V = 100000

_NW = 32
_ROWS = _N * _B
_RPW = _ROWS // _NW
_CHUNK = 64
_NCH = _RPW // _CHUNK


def _sc_gather(mean_flat, std_flat, idx3d):
    mesh = plsc.VectorSubcoreMesh(
        core_axis_name="c", subcore_axis_name="s", num_cores=2, num_subcores=16
    )

    @functools.partial(
        pl.kernel,
        out_type=(
            jax.ShapeDtypeStruct((_NW, _NCH, _CHUNK, _D), jnp.float32),
            jax.ShapeDtypeStruct((_NW, _NCH, _CHUNK, _D), jnp.float32),
        ),
        mesh=mesh,
        scratch_types=(
            pltpu.VMEM((_NCH, _CHUNK), jnp.int32),
            pltpu.VMEM((_NCH, _CHUNK, _D), jnp.float32),
            pltpu.VMEM((_NCH, _CHUNK, _D), jnp.float32),
            pltpu.SemaphoreType.DMA,
        ),
        compiler_params=pltpu.CompilerParams(use_tc_tiling_on_sc=False),
    )
    def k(mean_hbm, std_hbm, idx_hbm, mu_out, sr_out, idx_v, mu_v, sr_v, sem):
        wid = lax.axis_index("s") * 2 + lax.axis_index("c")
        pltpu.sync_copy(idx_hbm.at[wid], idx_v)
        handles = []
        for c in range(_NCH):
            handles.append(pltpu.async_copy(mean_hbm.at[idx_v.at[c]], mu_v.at[c], sem))
            handles.append(pltpu.async_copy(std_hbm.at[idx_v.at[c]], sr_v.at[c], sem))
        for h in handles:
            h.wait()
        pltpu.sync_copy(mu_v, mu_out.at[wid])
        pltpu.sync_copy(sr_v, sr_out.at[wid])

    return k(mean_flat, std_flat, idx3d)


def _tc_body(mu_ref, sr_ref, v_ref, reg_ref, cat_ref, la_ref, lab_ref,
             inf_ref, rew_ref):
    f32 = jnp.float32
    best = la_ref[0, 0]
    sel = jnp.int32(0)
    for p in range(1, 5):
        cur = la_ref[0, p]
        take = cur > best
        sel = jnp.where(take, jnp.int32(p), sel)
        best = jnp.where(take, cur, best)
    w = [(sel == p).astype(f32) for p in range(5)]

    mu = jnp.transpose(mu_ref[...], (0, 2, 1))
    srw = jnp.transpose(sr_ref[...], (0, 2, 1))
    vt = v_ref[...]
    emb = mu + jnp.log(1.0 + jnp.exp(srw)) * vt[None] * 0.01

    reg = reg_ref[...]
    cat = cat_ref[...]

    rows = []
    for o in range(2):
        s = w[0] * (jnp.sum(reg[:, :, 0, o, :], axis=1)
                    + jnp.sum(reg[:, :, 0, o, :], axis=0))
        s = s + 0.5 * w[2] * (jnp.sum(reg[:, :, 2, o, :], axis=1)
                              + jnp.sum(reg[:, :, 2, o, :], axis=0))
        s = s + 0.5 * w[3] * (jnp.sum(reg[:, :, 3, o, :], axis=1)
                              + jnp.sum(reg[:, :, 3, o, :], axis=0))
        s = s + w[4] * (jnp.sum(cat[:, :, o, :_D], axis=1)
                        + jnp.sum(cat[:, :, o, _D:], axis=0))
        rows.append(jnp.sum(emb * s[:, :, None], axis=(0, 1)))
    inf_ref[...] = jnp.concatenate([rows[0][None], rows[1][None]], axis=0)

    @pl.when(sel == 1)
    def _mult_quad():
        q0 = jnp.zeros((_B,), f32)
        q1 = jnp.zeros((_B,), f32)
        for i in range(_N):
            prod = emb[i][None] * emb
            q0 = q0 + jnp.sum(prod * reg[i, :, 1, 0, :][:, :, None], axis=(0, 1))
            q1 = q1 + jnp.sum(prod * reg[i, :, 1, 1, :][:, :, None], axis=(0, 1))
        inf_ref[...] += jnp.concatenate([q0[None], q1[None]], axis=0)

    @pl.when((sel == 2) | (sel == 3))
    def _abs_quad():
        a0 = 0.5 * (w[2] * reg[:, :, 2, 0, :] - w[3] * reg[:, :, 3, 0, :])
        a1 = 0.5 * (w[2] * reg[:, :, 2, 1, :] - w[3] * reg[:, :, 3, 1, :])
        q0 = jnp.zeros((_B,), f32)
        q1 = jnp.zeros((_B,), f32)
        for i in range(_N):
            ad = jnp.abs(emb[i][None] - emb)
            q0 = q0 + jnp.sum(ad * a0[i][:, :, None], axis=(0, 1))
            q1 = q1 + jnp.sum(ad * a1[i][:, :, None], axis=(0, 1))
        inf_ref[...] += jnp.concatenate([q0[None], q1[None]], axis=0)

    inf = inf_ref[...]
    lab = lab_ref[...]
    win0 = inf[0] >= inf[1]
    rew_ref[...] = jnp.reshape(jnp.sum(jnp.where(win0, lab[0], lab[1])), (1, 1))


def _tc_compute(mu, sr, v_t, fc_reg, fc_cat, log_alpha, label_t, interpret=False):
    return pl.pallas_call(
        _tc_body,
        out_shape=(
            jax.ShapeDtypeStruct((2, _B), jnp.float32),
            jax.ShapeDtypeStruct((1, 1), jnp.float32),
        ),
        in_specs=[
            pl.BlockSpec(memory_space=pltpu.VMEM),
            pl.BlockSpec(memory_space=pltpu.VMEM),
            pl.BlockSpec(memory_space=pltpu.VMEM),
            pl.BlockSpec(memory_space=pltpu.VMEM),
            pl.BlockSpec(memory_space=pltpu.VMEM),
            pl.BlockSpec(memory_space=pltpu.SMEM),
            pl.BlockSpec(memory_space=pltpu.VMEM),
        ],
        out_specs=(
            pl.BlockSpec(memory_space=pltpu.VMEM),
            pl.BlockSpec(memory_space=pltpu.VMEM),
        ),
        interpret=interpret,
    )(mu, sr, v_t, fc_reg, fc_cat, log_alpha, label_t)


def kernel(features, label, emb_mean, emb_std, fc_reg, fc_cat, log_alpha,
           rand_array):
    idx = (features.astype(jnp.int32)
           + (jnp.arange(_N, dtype=jnp.int32) * _V)[:, None])
    idx3d = idx.reshape(_NW, _NCH, _CHUNK)
    mu_g, sr_g = _sc_gather(
        emb_mean.reshape(_N * _V, _D), emb_std.reshape(_N * _V, _D), idx3d
    )
    mu_g = mu_g.reshape(_N, _B, _D)
    sr_g = sr_g.reshape(_N, _B, _D)
    v_t = rand_array[: _B * _D].reshape(_B, _D).T
    inf_t, rew = _tc_compute(mu_g, sr_g, v_t, fc_reg, fc_cat, log_alpha,
                             label.T)
    return inf_t.T, rew[0, 0]


# speedup vs baseline: 2.9413x; 2.9413x over previous
import functools

import jax
import jax.numpy as jnp
from jax import lax
from jax.experimental import pallas as pl
from jax.experimental.pallas import tpu as pltpu
from jax.experimental.pallas import tpu_sc as plsc

_N = 22
_B = 1024
_D = 32
_V = 100000

_NW = 32
_ROWS = _N * _B
_RPW = _ROWS // _NW
_CHUNK = 64
_NCH = _RPW // _CHUNK


def _sc_gather(mean3, std3, tidx, tsub):
    mesh = plsc.VectorSubcoreMesh(
        core_axis_name="c", subcore_axis_name="s", num_cores=2, num_subcores=16
    )
    n_chunks = _B // _CHUNK

    @functools.partial(
        pl.kernel,
        out_type=(
            jax.ShapeDtypeStruct((_N, _D, _B), jnp.float32),
            jax.ShapeDtypeStruct((_N, _D, _B), jnp.float32),
        ),
        mesh=mesh,
        scratch_types=(
            pltpu.VMEM((_B,), jnp.int32),
            pltpu.VMEM((_B,), jnp.int32),
            pltpu.VMEM((16, 8, _D), jnp.float32),
            pltpu.VMEM((16, 8, _D), jnp.float32),
            pltpu.VMEM((16, 8, _D), jnp.float32),
            pltpu.VMEM((16, 8, _D), jnp.float32),
            pltpu.VMEM((_D, _B // 2), jnp.float32),
            pltpu.VMEM((_D, _B // 2), jnp.float32),
            pltpu.SemaphoreType.DMA,
            pltpu.SemaphoreType.DMA,
        ),
        compiler_params=pltpu.CompilerParams(needs_layout_passes=False),
    )
    def k(mean_hbm, std_hbm, tidx_hbm, tsub_hbm, mu_out, sr_out,
          ti_v, ts_v, mu_t0, mu_t1, sr_t0, sr_t1, mu_s, sr_s, sem_a, sem_b):
        wid = lax.axis_index("s") * 2 + lax.axis_index("c")
        n_half = 64

        def fire(c, mu_t, sr_t, sem):
            t16 = ti_v[pl.ds(c * 16, 16)]
            for l in range(16):
                pltpu.async_copy(mean_hbm.at[t16[l]], mu_t.at[l], sem)
                pltpu.async_copy(std_hbm.at[t16[l]], sr_t.at[l], sem)

        def drain(mu_t, sr_t, sem):
            for l in range(16):
                pltpu.make_async_copy(mean_hbm.at[0], mu_t.at[l], sem).wait()
                pltpu.make_async_copy(std_hbm.at[0], sr_t.at[l], sem).wait()

        def select(c, mu_t, sr_t):
            jv = jax.lax.iota(jnp.int32, 16)
            sub = ts_v[pl.ds(c * 16, 16)]
            off = (c % 32) * 16
            for d in range(_D):
                dvec = jnp.full((16,), d, jnp.int32)
                mu_s[d, pl.ds(off, 16)] = plsc.load_gather(mu_t, [jv, sub, dvec])
                sr_s[d, pl.ds(off, 16)] = plsc.load_gather(sr_t, [jv, sub, dvec])

        def flush(part):
            pltpu.sync_copy(mu_s, mu_out.at[wid, :, pl.ds(part * (_B // 2), _B // 2)])
            pltpu.sync_copy(sr_s, sr_out.at[wid, :, pl.ds(part * (_B // 2), _B // 2)])

        @pl.when(wid < _N)
        def _work():
            pltpu.sync_copy(tidx_hbm.at[wid], ti_v)
            pltpu.sync_copy(tsub_hbm.at[wid], ts_v)
            fire(0, mu_t0, sr_t0, sem_a)

            def body(h, _):
                c0 = h * 2
                fire(c0 + 1, mu_t1, sr_t1, sem_b)
                drain(mu_t0, sr_t0, sem_a)
                select(c0, mu_t0, sr_t0)

                @pl.when(h < n_half // 2 - 1)
                def _():
                    fire(c0 + 2, mu_t0, sr_t0, sem_a)

                drain(mu_t1, sr_t1, sem_b)
                select(c0 + 1, mu_t1, sr_t1)

                @pl.when(h == 15)
                def _():
                    flush(0)

                return 0

            lax.fori_loop(0, n_half // 2, body, 0)
            flush(1)

    return k(mean3, std3, tidx, tsub)


def _tc_body(mu_ref, sr_ref, v_ref, reg_ref, cat_ref, la_ref, lab_ref,
             inf_ref, rew_ref):
    f32 = jnp.float32
    best = la_ref[0, 0]
    sel = jnp.int32(0)
    for p in range(1, 5):
        cur = la_ref[0, p]
        take = cur > best
        sel = jnp.where(take, jnp.int32(p), sel)
        best = jnp.where(take, cur, best)
    w = [(sel == p).astype(f32) for p in range(5)]

    mu = mu_ref[...]
    srw = sr_ref[...]
    vt = v_ref[...]
    emb = mu + jnp.log(1.0 + jnp.exp(srw)) * vt[None] * 0.01

    reg = reg_ref[...]
    cat = cat_ref[...]

    rows = []
    for o in range(2):
        s = w[0] * (jnp.sum(reg[:, :, 0, o, :], axis=1)
                    + jnp.sum(reg[:, :, 0, o, :], axis=0))
        s = s + 0.5 * w[2] * (jnp.sum(reg[:, :, 2, o, :], axis=1)
                              + jnp.sum(reg[:, :, 2, o, :], axis=0))
        s = s + 0.5 * w[3] * (jnp.sum(reg[:, :, 3, o, :], axis=1)
                              + jnp.sum(reg[:, :, 3, o, :], axis=0))
        s = s + w[4] * (jnp.sum(cat[:, :, o, :_D], axis=1)
                        + jnp.sum(cat[:, :, o, _D:], axis=0))
        rows.append(jnp.sum(emb * s[:, :, None], axis=(0, 1)))
    inf_ref[...] = jnp.concatenate([rows[0][None], rows[1][None]], axis=0)

    @pl.when(sel == 1)
    def _mult_quad():
        q0 = jnp.zeros((_B,), f32)
        q1 = jnp.zeros((_B,), f32)
        for i in range(_N):
            prod = emb[i][None] * emb
            q0 = q0 + jnp.sum(prod * reg[i, :, 1, 0, :][:, :, None], axis=(0, 1))
            q1 = q1 + jnp.sum(prod * reg[i, :, 1, 1, :][:, :, None], axis=(0, 1))
        inf_ref[...] += jnp.concatenate([q0[None], q1[None]], axis=0)

    @pl.when((sel == 2) | (sel == 3))
    def _abs_quad():
        a0 = 0.5 * (w[2] * reg[:, :, 2, 0, :] - w[3] * reg[:, :, 3, 0, :])
        a1 = 0.5 * (w[2] * reg[:, :, 2, 1, :] - w[3] * reg[:, :, 3, 1, :])
        q0 = jnp.zeros((_B,), f32)
        q1 = jnp.zeros((_B,), f32)
        for i in range(_N):
            ad = jnp.abs(emb[i][None] - emb)
            q0 = q0 + jnp.sum(ad * a0[i][:, :, None], axis=(0, 1))
            q1 = q1 + jnp.sum(ad * a1[i][:, :, None], axis=(0, 1))
        inf_ref[...] += jnp.concatenate([q0[None], q1[None]], axis=0)

    inf = inf_ref[...]
    lab = lab_ref[...]
    win0 = inf[0] >= inf[1]
    rew_ref[...] = jnp.reshape(jnp.sum(jnp.where(win0, lab[0], lab[1])), (1, 1))


def _tc_compute(mu, sr, v_t, fc_reg, fc_cat, log_alpha, label_t, interpret=False):
    return pl.pallas_call(
        _tc_body,
        out_shape=(
            jax.ShapeDtypeStruct((2, _B), jnp.float32),
            jax.ShapeDtypeStruct((1, 1), jnp.float32),
        ),
        in_specs=[
            pl.BlockSpec(memory_space=pltpu.VMEM),
            pl.BlockSpec(memory_space=pltpu.VMEM),
            pl.BlockSpec(memory_space=pltpu.VMEM),
            pl.BlockSpec(memory_space=pltpu.VMEM),
            pl.BlockSpec(memory_space=pltpu.VMEM),
            pl.BlockSpec(memory_space=pltpu.SMEM),
            pl.BlockSpec(memory_space=pltpu.VMEM),
        ],
        out_specs=(
            pl.BlockSpec(memory_space=pltpu.VMEM),
            pl.BlockSpec(memory_space=pltpu.VMEM),
        ),
        interpret=interpret,
    )(mu, sr, v_t, fc_reg, fc_cat, log_alpha, label_t)


def kernel(features, label, emb_mean, emb_std, fc_reg, fc_cat, log_alpha,
           rand_array):
    idx = (features.astype(jnp.int32)
           + (jnp.arange(_N, dtype=jnp.int32) * _V)[:, None])
    tidx = idx // 8
    tsub = idx - tidx * 8
    mu_g, sr_g = _sc_gather(
        emb_mean.reshape(_N * _V // 8, 8, _D),
        emb_std.reshape(_N * _V // 8, 8, _D), tidx, tsub
    )
    v_t = rand_array[: _B * _D].reshape(_B, _D).T
    inf_t, rew = _tc_compute(mu_g, sr_g, v_t, fc_reg, fc_cat, log_alpha,
                             label.T)
    return inf_t.T, rew[0, 0]


# speedup vs baseline: 4.2877x; 1.4578x over previous
import functools

import jax
import jax.numpy as jnp
from jax import lax
from jax.experimental import pallas as pl
from jax.experimental.pallas import tpu as pltpu
from jax.experimental.pallas import tpu_sc as plsc

_N = 22
_B = 1024
_D = 32
_V = 100000

_NW = 32
_ROWS = _N * _B
_RPW = _ROWS // _NW
_CHUNK = 64
_NCH = _RPW // _CHUNK


def _sc_gather(mean_t, std_t, feat):
    mesh = plsc.VectorSubcoreMesh(
        core_axis_name="c", subcore_axis_name="s", num_cores=2, num_subcores=16
    )
    n_chunks = _B // _CHUNK

    @functools.partial(
        pl.kernel,
        out_type=(
            jax.ShapeDtypeStruct((_N, _D, _B), jnp.float32),
            jax.ShapeDtypeStruct((_N, _D, _B), jnp.float32),
        ),
        mesh=mesh,
        scratch_types=(
            pltpu.VMEM((_B + 16,), jnp.int32),
            pltpu.VMEM((8, _D, 128), jnp.float32),
            pltpu.VMEM((8, _D, 128), jnp.float32),
            pltpu.VMEM((_D, _B // 2), jnp.float32),
            pltpu.VMEM((_D, _B // 2), jnp.float32),
            pltpu.SemaphoreType.DMA((8,)),
            pltpu.SemaphoreType.DMA((8,)),
        ),
        compiler_params=pltpu.CompilerParams(
            needs_layout_passes=False, use_tc_tiling_on_sc=True
        ),
    )
    def k(mean_hbm, std_hbm, feat_hbm, mu_out, sr_out,
          ti_v, mu_r, sr_r, mu_s, sr_s, sem_mu, sem_sr):
        wid = lax.axis_index("s") * 2 + lax.axis_index("c")
        n_rounds = _B // 8

        def flush(part):
            pltpu.sync_copy(mu_s, mu_out.at[wid, :, pl.ds(part * (_B // 2), _B // 2)])
            pltpu.sync_copy(sr_s, sr_out.at[wid, :, pl.ds(part * (_B // 2), _B // 2)])

        def fire(s, v):
            off = pl.multiple_of((v // 128) * 128, 128)
            pltpu.async_copy(mean_hbm.at[wid, :, pl.ds(off, 128)],
                             mu_r.at[s], sem_mu.at[s])
            pltpu.async_copy(std_hbm.at[wid, :, pl.ds(off, 128)],
                             sr_r.at[s], sem_sr.at[s])

        @pl.when(wid < _N)
        def _work():
            pltpu.sync_copy(feat_hbm.at[wid], ti_v.at[pl.ds(0, _B)])
            v16 = ti_v[pl.ds(0, 16)]
            for s in range(8):
                fire(s, v16[s])

            def body(kk, _):
                v16c = ti_v[pl.ds(kk * 8, 16)]
                v16n = ti_v[pl.ds(kk * 8 + 8, 16)]
                dlo = jax.lax.iota(jnp.int32, 16)
                dhi = dlo + 16
                for s in range(8):
                    pltpu.make_async_copy(mean_hbm.at[0, :, pl.ds(0, 128)],
                                          mu_r.at[s], sem_mu.at[s]).wait()
                    pltpu.make_async_copy(std_hbm.at[0, :, pl.ds(0, 128)],
                                          sr_r.at[s], sem_sr.at[s]).wait()
                    b = (kk * 8 + s) % (_B // 2)
                    bvec = jnp.full((16,), b, jnp.int32)
                    svec = jnp.full((16,), s, jnp.int32)
                    lvec = jnp.full((16,), v16c[s] % 128, jnp.int32)
                    plsc.store_scatter(mu_s, [dlo, bvec],
                                       plsc.load_gather(mu_r, [svec, dlo, lvec]))
                    plsc.store_scatter(mu_s, [dhi, bvec],
                                       plsc.load_gather(mu_r, [svec, dhi, lvec]))
                    plsc.store_scatter(sr_s, [dlo, bvec],
                                       plsc.load_gather(sr_r, [svec, dlo, lvec]))
                    plsc.store_scatter(sr_s, [dhi, bvec],
                                       plsc.load_gather(sr_r, [svec, dhi, lvec]))

                    @pl.when(kk < n_rounds - 1)
                    def _():
                        fire(s, v16n[s])

                @pl.when(kk == n_rounds // 2 - 1)
                def _():
                    flush(0)

                return 0

            lax.fori_loop(0, n_rounds, body, 0)
            flush(1)

    return k(mean_t, std_t, feat)


def _tc_body(mu_ref, sr_ref, v_ref, reg_ref, cat_ref, la_ref, lab_ref,
             inf_ref, rew_ref):
    f32 = jnp.float32
    best = la_ref[0, 0]
    sel = jnp.int32(0)
    for p in range(1, 5):
        cur = la_ref[0, p]
        take = cur > best
        sel = jnp.where(take, jnp.int32(p), sel)
        best = jnp.where(take, cur, best)
    w = [(sel == p).astype(f32) for p in range(5)]

    mu = mu_ref[...]
    srw = sr_ref[...]
    vt = v_ref[...]
    emb = mu + jnp.log(1.0 + jnp.exp(srw)) * vt[None] * 0.01

    reg = reg_ref[...]
    cat = cat_ref[...]

    rows = []
    for o in range(2):
        s = w[0] * (jnp.sum(reg[:, :, 0, o, :], axis=1)
                    + jnp.sum(reg[:, :, 0, o, :], axis=0))
        s = s + 0.5 * w[2] * (jnp.sum(reg[:, :, 2, o, :], axis=1)
                              + jnp.sum(reg[:, :, 2, o, :], axis=0))
        s = s + 0.5 * w[3] * (jnp.sum(reg[:, :, 3, o, :], axis=1)
                              + jnp.sum(reg[:, :, 3, o, :], axis=0))
        s = s + w[4] * (jnp.sum(cat[:, :, o, :_D], axis=1)
                        + jnp.sum(cat[:, :, o, _D:], axis=0))
        rows.append(jnp.sum(emb * s[:, :, None], axis=(0, 1)))
    inf_ref[...] = jnp.concatenate([rows[0][None], rows[1][None]], axis=0)

    @pl.when(sel == 1)
    def _mult_quad():
        q0 = jnp.zeros((_B,), f32)
        q1 = jnp.zeros((_B,), f32)
        for i in range(_N):
            prod = emb[i][None] * emb
            q0 = q0 + jnp.sum(prod * reg[i, :, 1, 0, :][:, :, None], axis=(0, 1))
            q1 = q1 + jnp.sum(prod * reg[i, :, 1, 1, :][:, :, None], axis=(0, 1))
        inf_ref[...] += jnp.concatenate([q0[None], q1[None]], axis=0)

    @pl.when((sel == 2) | (sel == 3))
    def _abs_quad():
        a0 = 0.5 * (w[2] * reg[:, :, 2, 0, :] - w[3] * reg[:, :, 3, 0, :])
        a1 = 0.5 * (w[2] * reg[:, :, 2, 1, :] - w[3] * reg[:, :, 3, 1, :])
        q0 = jnp.zeros((_B,), f32)
        q1 = jnp.zeros((_B,), f32)
        for i in range(_N):
            ad = jnp.abs(emb[i][None] - emb)
            q0 = q0 + jnp.sum(ad * a0[i][:, :, None], axis=(0, 1))
            q1 = q1 + jnp.sum(ad * a1[i][:, :, None], axis=(0, 1))
        inf_ref[...] += jnp.concatenate([q0[None], q1[None]], axis=0)

    inf = inf_ref[...]
    lab = lab_ref[...]
    win0 = inf[0] >= inf[1]
    rew_ref[...] = jnp.reshape(jnp.sum(jnp.where(win0, lab[0], lab[1])), (1, 1))


def _tc_compute(mu, sr, v_t, fc_reg, fc_cat, log_alpha, label_t, interpret=False):
    return pl.pallas_call(
        _tc_body,
        out_shape=(
            jax.ShapeDtypeStruct((2, _B), jnp.float32),
            jax.ShapeDtypeStruct((1, 1), jnp.float32),
        ),
        in_specs=[
            pl.BlockSpec(memory_space=pltpu.VMEM),
            pl.BlockSpec(memory_space=pltpu.VMEM),
            pl.BlockSpec(memory_space=pltpu.VMEM),
            pl.BlockSpec(memory_space=pltpu.VMEM),
            pl.BlockSpec(memory_space=pltpu.VMEM),
            pl.BlockSpec(memory_space=pltpu.SMEM),
            pl.BlockSpec(memory_space=pltpu.VMEM),
        ],
        out_specs=(
            pl.BlockSpec(memory_space=pltpu.VMEM),
            pl.BlockSpec(memory_space=pltpu.VMEM),
        ),
        interpret=interpret,
    )(mu, sr, v_t, fc_reg, fc_cat, log_alpha, label_t)


def kernel(features, label, emb_mean, emb_std, fc_reg, fc_cat, log_alpha,
           rand_array):
    mu_g, sr_g = _sc_gather(
        emb_mean.transpose(0, 2, 1), emb_std.transpose(0, 2, 1),
        features.astype(jnp.int32)
    )
    v_t = rand_array[: _B * _D].reshape(_B, _D).T
    inf_t, rew = _tc_compute(mu_g, sr_g, v_t, fc_reg, fc_cat, log_alpha,
                             label.T)
    return inf_t.T, rew[0, 0]


# speedup vs baseline: 4.7540x; 1.1087x over previous
import functools

import jax
import jax.numpy as jnp
from jax import lax
from jax.experimental import pallas as pl
from jax.experimental.pallas import tpu as pltpu
from jax.experimental.pallas import tpu_sc as plsc

_N = 22
_B = 1024
_D = 32
_V = 100000

_NW = 32
_ROWS = _N * _B
_RPW = _ROWS // _NW
_CHUNK = 64
_NCH = _RPW // _CHUNK


def _sc_gather(mean_t, std_t, fetch_list, flags, lanes, cols):
    mesh = plsc.VectorSubcoreMesh(
        core_axis_name="c", subcore_axis_name="s", num_cores=2, num_subcores=16
    )
    n_chunks = _B // _CHUNK

    @functools.partial(
        pl.kernel,
        out_type=(
            jax.ShapeDtypeStruct((_N, _D, _B), jnp.float32),
            jax.ShapeDtypeStruct((_N, _D, _B), jnp.float32),
        ),
        mesh=mesh,
        scratch_types=(
            pltpu.VMEM((_B + 16,), jnp.int32),
            pltpu.VMEM((_B,), jnp.int32),
            pltpu.VMEM((_B,), jnp.int32),
            pltpu.VMEM((_B,), jnp.int32),
            pltpu.VMEM((6, _D, 128), jnp.float32),
            pltpu.VMEM((6, _D, 128), jnp.float32),
            pltpu.VMEM((_D, _B), jnp.float32),
            pltpu.VMEM((_D, _B), jnp.float32),
            pltpu.SemaphoreType.DMA((6,)),
            pltpu.SemaphoreType.DMA((6,)),
        ),
        compiler_params=pltpu.CompilerParams(
            needs_layout_passes=False, use_tc_tiling_on_sc=True
        ),
    )
    def k(mean_hbm, std_hbm, fl_hbm, flag_hbm, lane_hbm, col_hbm,
          mu_out, sr_out, fl_v, flag_v, lane_v, col_v, mu_r, sr_r,
          mu_s, sr_s, sem_mu, sem_sr):
        wid = lax.axis_index("s") * 2 + lax.axis_index("c")

        def fire(ring, chunk):
            off = pl.multiple_of(chunk * 128, 128)
            pltpu.async_copy(mean_hbm.at[wid, :, pl.ds(off, 128)],
                             mu_r.at[ring], sem_mu.at[ring])
            pltpu.async_copy(std_hbm.at[wid, :, pl.ds(off, 128)],
                             sr_r.at[ring], sem_sr.at[ring])

        def wait_slot(ring):
            pltpu.make_async_copy(mean_hbm.at[0, :, pl.ds(0, 128)],
                                  mu_r.at[ring], sem_mu.at[ring]).wait()
            pltpu.make_async_copy(std_hbm.at[0, :, pl.ds(0, 128)],
                                  sr_r.at[ring], sem_sr.at[ring]).wait()

        @pl.when(wid < _N)
        def _work():
            pltpu.sync_copy(fl_hbm.at[wid], fl_v)
            pltpu.sync_copy(flag_hbm.at[wid], flag_v)
            pltpu.sync_copy(lane_hbm.at[wid], lane_v)
            pltpu.sync_copy(col_hbm.at[wid], col_v)
            f16 = fl_v[pl.ds(0, 16)]
            for s in range(6):
                fire(s, f16[s])
            wait_slot(0)

            def body(kk, slot):
                flg16 = flag_v[pl.ds(kk * 8, 16)]
                lan16 = lane_v[pl.ds(kk * 8, 16)]
                col16 = col_v[pl.ds(kk * 8, 16)]
                dlo = jax.lax.iota(jnp.int32, 16)
                dhi = dlo + 16
                for s in range(8):
                    slot = slot + flg16[s]

                    @pl.when(flg16[s] == 1)
                    def _(slot=slot):
                        nxt = plsc.load_gather(
                            fl_v, [jnp.full((16,), slot + 5, jnp.int32)])[0]
                        fire((slot + 5) % 6, nxt)
                        wait_slot(slot % 6)

                    svec = jnp.full((16,), slot % 6, jnp.int32)
                    lvec = jnp.full((16,), lan16[s], jnp.int32)
                    bvec = jnp.full((16,), col16[s], jnp.int32)
                    plsc.store_scatter(mu_s, [dlo, bvec],
                                       plsc.load_gather(mu_r, [svec, dlo, lvec]))
                    plsc.store_scatter(mu_s, [dhi, bvec],
                                       plsc.load_gather(mu_r, [svec, dhi, lvec]))
                    plsc.store_scatter(sr_s, [dlo, bvec],
                                       plsc.load_gather(sr_r, [svec, dlo, lvec]))
                    plsc.store_scatter(sr_s, [dhi, bvec],
                                       plsc.load_gather(sr_r, [svec, dhi, lvec]))
                return slot

            slot = lax.fori_loop(0, _B // 8, body, jnp.int32(0))
            for i in range(1, 6):
                wait_slot((slot + i) % 6)
            pltpu.sync_copy(mu_s, mu_out.at[wid])
            pltpu.sync_copy(sr_s, sr_out.at[wid])

    return k(mean_t, std_t, fetch_list, flags, lanes, cols)


def _tc_body(mu_ref, sr_ref, v_ref, reg_ref, cat_ref, la_ref, lab_ref,
             inf_ref, rew_ref):
    f32 = jnp.float32
    best = la_ref[0, 0]
    sel = jnp.int32(0)
    for p in range(1, 5):
        cur = la_ref[0, p]
        take = cur > best
        sel = jnp.where(take, jnp.int32(p), sel)
        best = jnp.where(take, cur, best)
    w = [(sel == p).astype(f32) for p in range(5)]

    mu = mu_ref[...]
    srw = sr_ref[...]
    vt = v_ref[...]
    emb = mu + jnp.log(1.0 + jnp.exp(srw)) * vt[None] * 0.01

    reg = reg_ref[...]
    cat = cat_ref[...]

    rows = []
    for o in range(2):
        s = w[0] * (jnp.sum(reg[:, :, 0, o, :], axis=1)
                    + jnp.sum(reg[:, :, 0, o, :], axis=0))
        s = s + 0.5 * w[2] * (jnp.sum(reg[:, :, 2, o, :], axis=1)
                              + jnp.sum(reg[:, :, 2, o, :], axis=0))
        s = s + 0.5 * w[3] * (jnp.sum(reg[:, :, 3, o, :], axis=1)
                              + jnp.sum(reg[:, :, 3, o, :], axis=0))
        s = s + w[4] * (jnp.sum(cat[:, :, o, :_D], axis=1)
                        + jnp.sum(cat[:, :, o, _D:], axis=0))
        rows.append(jnp.sum(emb * s[:, :, None], axis=(0, 1)))
    inf_ref[...] = jnp.concatenate([rows[0][None], rows[1][None]], axis=0)

    @pl.when(sel == 1)
    def _mult_quad():
        q0 = jnp.zeros((_B,), f32)
        q1 = jnp.zeros((_B,), f32)
        for i in range(_N):
            prod = emb[i][None] * emb
            q0 = q0 + jnp.sum(prod * reg[i, :, 1, 0, :][:, :, None], axis=(0, 1))
            q1 = q1 + jnp.sum(prod * reg[i, :, 1, 1, :][:, :, None], axis=(0, 1))
        inf_ref[...] += jnp.concatenate([q0[None], q1[None]], axis=0)

    @pl.when((sel == 2) | (sel == 3))
    def _abs_quad():
        a0 = 0.5 * (w[2] * reg[:, :, 2, 0, :] - w[3] * reg[:, :, 3, 0, :])
        a1 = 0.5 * (w[2] * reg[:, :, 2, 1, :] - w[3] * reg[:, :, 3, 1, :])
        q0 = jnp.zeros((_B,), f32)
        q1 = jnp.zeros((_B,), f32)
        for i in range(_N):
            ad = jnp.abs(emb[i][None] - emb)
            q0 = q0 + jnp.sum(ad * a0[i][:, :, None], axis=(0, 1))
            q1 = q1 + jnp.sum(ad * a1[i][:, :, None], axis=(0, 1))
        inf_ref[...] += jnp.concatenate([q0[None], q1[None]], axis=0)

    inf = inf_ref[...]
    lab = lab_ref[...]
    win0 = inf[0] >= inf[1]
    rew_ref[...] = jnp.reshape(jnp.sum(jnp.where(win0, lab[0], lab[1])), (1, 1))


def _tc_compute(mu, sr, v_t, fc_reg, fc_cat, log_alpha, label_t, interpret=False):
    return pl.pallas_call(
        _tc_body,
        out_shape=(
            jax.ShapeDtypeStruct((2, _B), jnp.float32),
            jax.ShapeDtypeStruct((1, 1), jnp.float32),
        ),
        in_specs=[
            pl.BlockSpec(memory_space=pltpu.VMEM),
            pl.BlockSpec(memory_space=pltpu.VMEM),
            pl.BlockSpec(memory_space=pltpu.VMEM),
            pl.BlockSpec(memory_space=pltpu.VMEM),
            pl.BlockSpec(memory_space=pltpu.VMEM),
            pl.BlockSpec(memory_space=pltpu.SMEM),
            pl.BlockSpec(memory_space=pltpu.VMEM),
        ],
        out_specs=(
            pl.BlockSpec(memory_space=pltpu.VMEM),
            pl.BlockSpec(memory_space=pltpu.VMEM),
        ),
        interpret=interpret,
    )(mu, sr, v_t, fc_reg, fc_cat, log_alpha, label_t)


def kernel(features, label, emb_mean, emb_std, fc_reg, fc_cat, log_alpha,
           rand_array):
    v = features.astype(jnp.int32)
    order = jnp.argsort(v, axis=1).astype(jnp.int32)
    vs = jnp.take_along_axis(v, order, axis=1)
    chunk = vs // 128
    flags = jnp.concatenate(
        [jnp.zeros((_N, 1), jnp.int32),
         (chunk[:, 1:] != chunk[:, :-1]).astype(jnp.int32)], axis=1)
    slots = jnp.cumsum(flags, axis=1)
    fetch_list = jnp.zeros((_N, _B + 16), jnp.int32).at[
        jnp.arange(_N, dtype=jnp.int32)[:, None], slots].set(chunk)
    mu_g, sr_g = _sc_gather(
        emb_mean.transpose(0, 2, 1), emb_std.transpose(0, 2, 1),
        fetch_list, flags, vs % 128, order
    )
    v_t = rand_array[: _B * _D].reshape(_B, _D).T
    inf_t, rew = _tc_compute(mu_g, sr_g, v_t, fc_reg, fc_cat, log_alpha,
                             label.T)
    return inf_t.T, rew[0, 0]


# speedup vs baseline: 4.7961x; 1.0088x over previous
import functools

import jax
import jax.numpy as jnp
from jax import lax
from jax.experimental import pallas as pl
from jax.experimental.pallas import tpu as pltpu
from jax.experimental.pallas import tpu_sc as plsc

_N = 22
_B = 1024
_D = 32
_V = 100000

_NW = 32
_ROWS = _N * _B
_RPW = _ROWS // _NW
_CHUNK = 64
_NCH = _RPW // _CHUNK


def _sc_gather(mean_t, std_t, fetch_list, flags, lanes, cols):
    mesh = plsc.VectorSubcoreMesh(
        core_axis_name="c", subcore_axis_name="s", num_cores=2, num_subcores=16
    )
    n_chunks = _B // _CHUNK

    @functools.partial(
        pl.kernel,
        out_type=(
            jax.ShapeDtypeStruct((_N, _D, _B), jnp.float32),
            jax.ShapeDtypeStruct((_N, _D, _B), jnp.float32),
        ),
        mesh=mesh,
        scratch_types=(
            pltpu.VMEM((_B + 16,), jnp.int32),
            pltpu.VMEM((_B,), jnp.int32),
            pltpu.VMEM((_B,), jnp.int32),
            pltpu.VMEM((_B,), jnp.int32),
            pltpu.VMEM((7, _D, 128), jnp.float32),
            pltpu.VMEM((7, _D, 128), jnp.float32),
            pltpu.VMEM((_D, _B), jnp.float32),
            pltpu.VMEM((_D, _B), jnp.float32),
            pltpu.SemaphoreType.DMA((7,)),
            pltpu.SemaphoreType.DMA((7,)),
        ),
        compiler_params=pltpu.CompilerParams(
            needs_layout_passes=False, use_tc_tiling_on_sc=True
        ),
    )
    def k(mean_hbm, std_hbm, fl_hbm, flag_hbm, lane_hbm, col_hbm,
          mu_out, sr_out, fl_v, flag_v, lane_v, col_v, mu_r, sr_r,
          mu_s, sr_s, sem_mu, sem_sr):
        wid = lax.axis_index("s") * 2 + lax.axis_index("c")

        def fire(ring, chunk):
            off = pl.multiple_of(chunk * 128, 128)
            for dt in range(4):
                ds8 = pl.ds(dt * 8, 8)
                pltpu.async_copy(mean_hbm.at[wid, ds8, pl.ds(off, 128)],
                                 mu_r.at[ring, ds8], sem_mu.at[ring])
                pltpu.async_copy(std_hbm.at[wid, ds8, pl.ds(off, 128)],
                                 sr_r.at[ring, ds8], sem_sr.at[ring])

        def wait_slot(ring):
            pltpu.make_async_copy(mean_hbm.at[0, :, pl.ds(0, 128)],
                                  mu_r.at[ring], sem_mu.at[ring]).wait()
            pltpu.make_async_copy(std_hbm.at[0, :, pl.ds(0, 128)],
                                  sr_r.at[ring], sem_sr.at[ring]).wait()

        @pl.when(wid < _N)
        def _work():
            pltpu.sync_copy(fl_hbm.at[wid], fl_v)
            pltpu.sync_copy(flag_hbm.at[wid], flag_v)
            pltpu.sync_copy(lane_hbm.at[wid], lane_v)
            pltpu.sync_copy(col_hbm.at[wid], col_v)
            f16 = fl_v[pl.ds(0, 16)]
            for s in range(7):
                fire(s, f16[s])
            wait_slot(0)

            def body(kk, slot):
                flg16 = flag_v[pl.ds(kk * 8, 16)]
                lan16 = lane_v[pl.ds(kk * 8, 16)]
                col16 = col_v[pl.ds(kk * 8, 16)]
                dlo = jax.lax.iota(jnp.int32, 16)
                dhi = dlo + 16
                for s in range(8):
                    slot = slot + flg16[s]

                    @pl.when(flg16[s] == 1)
                    def _(slot=slot):
                        nxt = plsc.load_gather(
                            fl_v, [jnp.full((16,), slot + 6, jnp.int32)])[0]
                        fire((slot + 6) % 7, nxt)
                        wait_slot(slot % 7)

                    svec = jnp.full((16,), slot % 7, jnp.int32)
                    lvec = jnp.full((16,), lan16[s], jnp.int32)
                    bvec = jnp.full((16,), col16[s], jnp.int32)
                    plsc.store_scatter(mu_s, [dlo, bvec],
                                       plsc.load_gather(mu_r, [svec, dlo, lvec]))
                    plsc.store_scatter(mu_s, [dhi, bvec],
                                       plsc.load_gather(mu_r, [svec, dhi, lvec]))
                    plsc.store_scatter(sr_s, [dlo, bvec],
                                       plsc.load_gather(sr_r, [svec, dlo, lvec]))
                    plsc.store_scatter(sr_s, [dhi, bvec],
                                       plsc.load_gather(sr_r, [svec, dhi, lvec]))
                return slot

            slot = lax.fori_loop(0, _B // 8, body, jnp.int32(0))
            for i in range(1, 7):
                wait_slot((slot + i) % 7)
            pltpu.sync_copy(mu_s, mu_out.at[wid])
            pltpu.sync_copy(sr_s, sr_out.at[wid])

    return k(mean_t, std_t, fetch_list, flags, lanes, cols)


def _tc_body(mu_ref, sr_ref, v_ref, reg_ref, cat_ref, la_ref, lab_ref,
             inf_ref, rew_ref):
    f32 = jnp.float32
    best = la_ref[0, 0]
    sel = jnp.int32(0)
    for p in range(1, 5):
        cur = la_ref[0, p]
        take = cur > best
        sel = jnp.where(take, jnp.int32(p), sel)
        best = jnp.where(take, cur, best)
    w = [(sel == p).astype(f32) for p in range(5)]

    mu = mu_ref[...]
    srw = sr_ref[...]
    vt = v_ref[...]
    emb = mu + jnp.log(1.0 + jnp.exp(srw)) * vt[None] * 0.01

    reg = reg_ref[...]
    cat = cat_ref[...]

    rows = []
    for o in range(2):
        s = w[0] * (jnp.sum(reg[:, :, 0, o, :], axis=1)
                    + jnp.sum(reg[:, :, 0, o, :], axis=0))
        s = s + 0.5 * w[2] * (jnp.sum(reg[:, :, 2, o, :], axis=1)
                              + jnp.sum(reg[:, :, 2, o, :], axis=0))
        s = s + 0.5 * w[3] * (jnp.sum(reg[:, :, 3, o, :], axis=1)
                              + jnp.sum(reg[:, :, 3, o, :], axis=0))
        s = s + w[4] * (jnp.sum(cat[:, :, o, :_D], axis=1)
                        + jnp.sum(cat[:, :, o, _D:], axis=0))
        rows.append(jnp.sum(emb * s[:, :, None], axis=(0, 1)))
    inf_ref[...] = jnp.concatenate([rows[0][None], rows[1][None]], axis=0)

    @pl.when(sel == 1)
    def _mult_quad():
        q0 = jnp.zeros((_B,), f32)
        q1 = jnp.zeros((_B,), f32)
        for i in range(_N):
            prod = emb[i][None] * emb
            q0 = q0 + jnp.sum(prod * reg[i, :, 1, 0, :][:, :, None], axis=(0, 1))
            q1 = q1 + jnp.sum(prod * reg[i, :, 1, 1, :][:, :, None], axis=(0, 1))
        inf_ref[...] += jnp.concatenate([q0[None], q1[None]], axis=0)

    @pl.when((sel == 2) | (sel == 3))
    def _abs_quad():
        a0 = 0.5 * (w[2] * reg[:, :, 2, 0, :] - w[3] * reg[:, :, 3, 0, :])
        a1 = 0.5 * (w[2] * reg[:, :, 2, 1, :] - w[3] * reg[:, :, 3, 1, :])
        q0 = jnp.zeros((_B,), f32)
        q1 = jnp.zeros((_B,), f32)
        for i in range(_N):
            ad = jnp.abs(emb[i][None] - emb)
            q0 = q0 + jnp.sum(ad * a0[i][:, :, None], axis=(0, 1))
            q1 = q1 + jnp.sum(ad * a1[i][:, :, None], axis=(0, 1))
        inf_ref[...] += jnp.concatenate([q0[None], q1[None]], axis=0)

    inf = inf_ref[...]
    lab = lab_ref[...]
    win0 = inf[0] >= inf[1]
    rew_ref[...] = jnp.reshape(jnp.sum(jnp.where(win0, lab[0], lab[1])), (1, 1))


def _tc_compute(mu, sr, v_t, fc_reg, fc_cat, log_alpha, label_t, interpret=False):
    return pl.pallas_call(
        _tc_body,
        out_shape=(
            jax.ShapeDtypeStruct((2, _B), jnp.float32),
            jax.ShapeDtypeStruct((1, 1), jnp.float32),
        ),
        in_specs=[
            pl.BlockSpec(memory_space=pltpu.VMEM),
            pl.BlockSpec(memory_space=pltpu.VMEM),
            pl.BlockSpec(memory_space=pltpu.VMEM),
            pl.BlockSpec(memory_space=pltpu.VMEM),
            pl.BlockSpec(memory_space=pltpu.VMEM),
            pl.BlockSpec(memory_space=pltpu.SMEM),
            pl.BlockSpec(memory_space=pltpu.VMEM),
        ],
        out_specs=(
            pl.BlockSpec(memory_space=pltpu.VMEM),
            pl.BlockSpec(memory_space=pltpu.VMEM),
        ),
        interpret=interpret,
    )(mu, sr, v_t, fc_reg, fc_cat, log_alpha, label_t)


def kernel(features, label, emb_mean, emb_std, fc_reg, fc_cat, log_alpha,
           rand_array):
    v = features.astype(jnp.int32)
    order = jnp.argsort(v, axis=1).astype(jnp.int32)
    vs = jnp.take_along_axis(v, order, axis=1)
    chunk = vs // 128
    flags = jnp.concatenate(
        [jnp.zeros((_N, 1), jnp.int32),
         (chunk[:, 1:] != chunk[:, :-1]).astype(jnp.int32)], axis=1)
    slots = jnp.cumsum(flags, axis=1)
    fetch_list = jnp.zeros((_N, _B + 16), jnp.int32).at[
        jnp.arange(_N, dtype=jnp.int32)[:, None], slots].set(chunk)
    mu_g, sr_g = _sc_gather(
        emb_mean.transpose(0, 2, 1), emb_std.transpose(0, 2, 1),
        fetch_list, flags, vs % 128, order
    )
    v_t = rand_array[: _B * _D].reshape(_B, _D).T
    inf_t, rew = _tc_compute(mu_g, sr_g, v_t, fc_reg, fc_cat, log_alpha,
                             label.T)
    return inf_t.T, rew[0, 0]


# speedup vs baseline: 4.8001x; 1.0008x over previous
import functools

import jax
import jax.numpy as jnp
from jax import lax
from jax.experimental import pallas as pl
from jax.experimental.pallas import tpu as pltpu
from jax.experimental.pallas import tpu_sc as plsc

_N = 22
_B = 1024
_D = 32
_V = 100000


def _sc_gather(mean_t, std_t, fetch_list, flags, lanes, cols):
    mesh = plsc.VectorSubcoreMesh(
        core_axis_name="c", subcore_axis_name="s", num_cores=2, num_subcores=16
    )

    @functools.partial(
        pl.kernel,
        out_type=(
            jax.ShapeDtypeStruct((_N, _D, _B), jnp.float32),
            jax.ShapeDtypeStruct((_N, _D, _B), jnp.float32),
        ),
        mesh=mesh,
        scratch_types=(
            pltpu.VMEM((_B + 16,), jnp.int32),
            pltpu.VMEM((_B,), jnp.int32),
            pltpu.VMEM((_B,), jnp.int32),
            pltpu.VMEM((_B,), jnp.int32),
            pltpu.VMEM((7, _D, 128), jnp.float32),
            pltpu.VMEM((7, _D, 128), jnp.float32),
            pltpu.VMEM((_D, _B), jnp.float32),
            pltpu.VMEM((_D, _B), jnp.float32),
            pltpu.SemaphoreType.DMA((7,)),
            pltpu.SemaphoreType.DMA((7,)),
        ),
        compiler_params=pltpu.CompilerParams(
            needs_layout_passes=False, use_tc_tiling_on_sc=True
        ),
    )
    def k(mean_hbm, std_hbm, fl_hbm, flag_hbm, lane_hbm, col_hbm,
          mu_out, sr_out, fl_v, flag_v, lane_v, col_v, mu_r, sr_r,
          mu_s, sr_s, sem_mu, sem_sr):
        wid = lax.axis_index("s") * 2 + lax.axis_index("c")

        def fire(ring, chunk):
            off = pl.multiple_of(chunk * 128, 128)
            for dt in range(4):
                ds8 = pl.ds(dt * 8, 8)
                pltpu.async_copy(mean_hbm.at[wid, ds8, pl.ds(off, 128)],
                                 mu_r.at[ring, ds8], sem_mu.at[ring])
                pltpu.async_copy(std_hbm.at[wid, ds8, pl.ds(off, 128)],
                                 sr_r.at[ring, ds8], sem_sr.at[ring])

        def wait_slot(ring):
            pltpu.make_async_copy(mean_hbm.at[0, :, pl.ds(0, 128)],
                                  mu_r.at[ring], sem_mu.at[ring]).wait()
            pltpu.make_async_copy(std_hbm.at[0, :, pl.ds(0, 128)],
                                  sr_r.at[ring], sem_sr.at[ring]).wait()

        @pl.when(wid < _N)
        def _work():
            pltpu.sync_copy(fl_hbm.at[wid], fl_v)
            pltpu.sync_copy(flag_hbm.at[wid], flag_v)
            pltpu.sync_copy(lane_hbm.at[wid], lane_v)
            pltpu.sync_copy(col_hbm.at[wid], col_v)
            f16 = fl_v[pl.ds(0, 16)]
            for s in range(7):
                fire(s, f16[s])
            wait_slot(0)

            def body(kk, slot):
                flg16 = flag_v[pl.ds(kk * 8, 16)]
                lan16 = lane_v[pl.ds(kk * 8, 16)]
                col16 = col_v[pl.ds(kk * 8, 16)]
                dlo = jax.lax.iota(jnp.int32, 16)
                dhi = dlo + 16
                for s in range(8):
                    slot = slot + flg16[s]

                    @pl.when(flg16[s] == 1)
                    def _(slot=slot):
                        nxt = plsc.load_gather(
                            fl_v, [jnp.full((16,), slot + 6, jnp.int32)])[0]
                        fire((slot + 6) % 7, nxt)
                        wait_slot(slot % 7)

                    svec = jnp.full((16,), slot % 7, jnp.int32)
                    lvec = jnp.full((16,), lan16[s], jnp.int32)
                    bvec = jnp.full((16,), col16[s], jnp.int32)
                    plsc.store_scatter(mu_s, [dlo, bvec],
                                       plsc.load_gather(mu_r, [svec, dlo, lvec]))
                    plsc.store_scatter(mu_s, [dhi, bvec],
                                       plsc.load_gather(mu_r, [svec, dhi, lvec]))
                    plsc.store_scatter(sr_s, [dlo, bvec],
                                       plsc.load_gather(sr_r, [svec, dlo, lvec]))
                    plsc.store_scatter(sr_s, [dhi, bvec],
                                       plsc.load_gather(sr_r, [svec, dhi, lvec]))
                return slot

            slot = lax.fori_loop(0, _B // 8, body, jnp.int32(0))
            for i in range(1, 7):
                wait_slot((slot + i) % 7)
            pltpu.sync_copy(mu_s, mu_out.at[wid])
            pltpu.sync_copy(sr_s, sr_out.at[wid])

    return k(mean_t, std_t, fetch_list, flags, lanes, cols)


def _tc_body(mu_ref, sr_ref, v_ref, reg_ref, cat_ref, la_ref, lab_ref,
             inf_ref, rew_ref):
    f32 = jnp.float32
    best = la_ref[0, 0]
    sel = jnp.int32(0)
    for p in range(1, 5):
        cur = la_ref[0, p]
        take = cur > best
        sel = jnp.where(take, jnp.int32(p), sel)
        best = jnp.where(take, cur, best)
    w = [(sel == p).astype(f32) for p in range(5)]

    mu = mu_ref[...]
    srw = sr_ref[...]
    vt = v_ref[...]
    emb = mu + jnp.log(1.0 + jnp.exp(srw)) * vt[None] * 0.01

    reg = reg_ref[...]
    cat = cat_ref[...]

    rows = []
    for o in range(2):
        s = w[0] * (jnp.sum(reg[:, :, 0, o, :], axis=1)
                    + jnp.sum(reg[:, :, 0, o, :], axis=0))
        s = s + 0.5 * w[2] * (jnp.sum(reg[:, :, 2, o, :], axis=1)
                              + jnp.sum(reg[:, :, 2, o, :], axis=0))
        s = s + 0.5 * w[3] * (jnp.sum(reg[:, :, 3, o, :], axis=1)
                              + jnp.sum(reg[:, :, 3, o, :], axis=0))
        s = s + w[4] * (jnp.sum(cat[:, :, o, :_D], axis=1)
                        + jnp.sum(cat[:, :, o, _D:], axis=0))
        rows.append(jnp.sum(emb * s[:, :, None], axis=(0, 1)))
    inf_ref[...] = jnp.concatenate([rows[0][None], rows[1][None]], axis=0)

    @pl.when(sel == 1)
    def _mult_quad():
        q0 = jnp.zeros((_B,), f32)
        q1 = jnp.zeros((_B,), f32)
        for i in range(_N):
            prod = emb[i][None] * emb
            q0 = q0 + jnp.sum(prod * reg[i, :, 1, 0, :][:, :, None], axis=(0, 1))
            q1 = q1 + jnp.sum(prod * reg[i, :, 1, 1, :][:, :, None], axis=(0, 1))
        inf_ref[...] += jnp.concatenate([q0[None], q1[None]], axis=0)

    @pl.when((sel == 2) | (sel == 3))
    def _abs_quad():
        a0 = 0.5 * (w[2] * reg[:, :, 2, 0, :] - w[3] * reg[:, :, 3, 0, :])
        a1 = 0.5 * (w[2] * reg[:, :, 2, 1, :] - w[3] * reg[:, :, 3, 1, :])
        q0 = jnp.zeros((_B,), f32)
        q1 = jnp.zeros((_B,), f32)
        for i in range(_N):
            ad = jnp.abs(emb[i][None] - emb)
            q0 = q0 + jnp.sum(ad * a0[i][:, :, None], axis=(0, 1))
            q1 = q1 + jnp.sum(ad * a1[i][:, :, None], axis=(0, 1))
        inf_ref[...] += jnp.concatenate([q0[None], q1[None]], axis=0)

    inf = inf_ref[...]
    lab = lab_ref[...]
    win0 = inf[0] >= inf[1]
    rew_ref[...] = jnp.reshape(jnp.sum(jnp.where(win0, lab[0], lab[1])), (1, 1))


def _tc_compute(mu, sr, v_t, fc_reg, fc_cat, log_alpha, label_t, interpret=False):
    return pl.pallas_call(
        _tc_body,
        out_shape=(
            jax.ShapeDtypeStruct((2, _B), jnp.float32),
            jax.ShapeDtypeStruct((1, 1), jnp.float32),
        ),
        in_specs=[
            pl.BlockSpec(memory_space=pltpu.VMEM),
            pl.BlockSpec(memory_space=pltpu.VMEM),
            pl.BlockSpec(memory_space=pltpu.VMEM),
            pl.BlockSpec(memory_space=pltpu.VMEM),
            pl.BlockSpec(memory_space=pltpu.VMEM),
            pl.BlockSpec(memory_space=pltpu.SMEM),
            pl.BlockSpec(memory_space=pltpu.VMEM),
        ],
        out_specs=(
            pl.BlockSpec(memory_space=pltpu.VMEM),
            pl.BlockSpec(memory_space=pltpu.VMEM),
        ),
        interpret=interpret,
    )(mu, sr, v_t, fc_reg, fc_cat, log_alpha, label_t)


def kernel(features, label, emb_mean, emb_std, fc_reg, fc_cat, log_alpha,
           rand_array):
    v = features.astype(jnp.int32)
    order = jnp.argsort(v, axis=1).astype(jnp.int32)
    vs = jnp.take_along_axis(v, order, axis=1)
    chunk = vs // 128
    flags = jnp.concatenate(
        [jnp.zeros((_N, 1), jnp.int32),
         (chunk[:, 1:] != chunk[:, :-1]).astype(jnp.int32)], axis=1)
    slots = jnp.cumsum(flags, axis=1)
    fetch_list = jnp.zeros((_N, _B + 16), jnp.int32).at[
        jnp.arange(_N, dtype=jnp.int32)[:, None], slots].set(chunk)
    mu_g, sr_g = _sc_gather(
        emb_mean.transpose(0, 2, 1), emb_std.transpose(0, 2, 1),
        fetch_list, flags, vs % 128, order
    )
    v_t = rand_array[: _B * _D].reshape(_B, _D).T
    inf_t, rew = _tc_compute(mu_g, sr_g, v_t, fc_reg, fc_cat, log_alpha,
                             label.T)
    return inf_t.T, rew[0, 0]
